# Initial kernel scaffold; baseline (speedup 1.0000x reference)
import jax
import jax.numpy as jnp
from jax.experimental import pallas as pl


def kernel(x, edge_index, batch, W1, b1, W2, b2, W3, b3):
    raise NotImplementedError("write your pallas kernel here")


# speedup vs baseline: 18.6127x; 18.6127x over previous
import jax
import jax.numpy as jnp
from jax import lax
from jax.experimental import pallas as pl
from jax.experimental.pallas import tpu as pltpu
from jax.experimental.pallas import tpu_sc as plsc

NSC = 2
NSUB = 16
FW = 16


def _zero_rows(buf, nrows, width):
  def row(r, carry):
    for j in range(width // 16):
      buf[r, pl.ds(j * 16, 16)] = jnp.zeros((16,), jnp.float32)
    return carry
  lax.fori_loop(0, nrows, row, 0)


def _sc_mesh():
  return plsc.VectorSubcoreMesh(core_axis_name="c", subcore_axis_name="s",
                                num_cores=NSC, num_subcores=NSUB)


def _make_deg(n, e, chunk=1000):
  epc = e // NSC
  ept = epc // NSUB
  nch = ept // chunk
  assert nch * chunk == ept
  rpt = n // NSUB
  zrows = 625
  assert rpt % zrows == 0

  def body(dst_hbm, deg_hbm, didx, ones_v, zbuf, acc):
    c = lax.axis_index("c")
    s = lax.axis_index("s")
    _zero_rows(zbuf, zrows, FW)
    def orow(r, carry):
      ones_v[r, pl.ds(0, 16)] = jnp.ones((16,), jnp.float32)
      return carry
    lax.fori_loop(0, chunk, orow, 0)
    for j in range(rpt // zrows):
      pltpu.sync_copy(zbuf, acc.at[pl.ds(s * rpt + j * zrows, zrows), :])
    plsc.subcore_barrier()
    base = c * epc + s * ept
    def ck(k, carry):
      pltpu.sync_copy(dst_hbm.at[pl.ds(base + k * chunk, chunk)], didx)
      pltpu.sync_copy(ones_v, acc.at[didx], add=True)
      return carry
    lax.fori_loop(0, nch, ck, 0)
    plsc.subcore_barrier()
    @pl.when(c == 0)
    def _():
      pltpu.sync_copy(acc.at[pl.ds(s * rpt, rpt), :],
                      deg_hbm.at[0, pl.ds(s * rpt, rpt), :])
    @pl.when(c == 1)
    def _():
      pltpu.sync_copy(acc.at[pl.ds(s * rpt, rpt), :],
                      deg_hbm.at[1, pl.ds(s * rpt, rpt), :])

  return pl.kernel(
      body,
      out_type=jax.ShapeDtypeStruct((NSC, n, FW), jnp.float32),
      mesh=_sc_mesh(),
      compiler_params=pltpu.CompilerParams(use_tc_tiling_on_sc=False),
      scratch_types=[
          pltpu.VMEM((chunk,), jnp.int32),
          pltpu.VMEM((chunk, FW), jnp.float32),
          pltpu.VMEM((625, FW), jnp.float32),
          pltpu.VMEM_SHARED((n, FW), jnp.float32),
      ],
  )


def _make_prop(n, e, chunk=1000):
  ept = e // NSUB
  nch = ept // chunk
  assert nch * chunk == ept
  rpt = n // NSUB
  zrows = 625
  assert rpt % zrows == 0

  def body(ylo_hbm, yhi_hbm, src_hbm, dst_hbm, z_hbm,
           sidx, didx, rows, zbuf, acc, sem):
    c = lax.axis_index("c")
    s = lax.axis_index("s")
    _zero_rows(zbuf, zrows, FW)
    for j in range(rpt // zrows):
      pltpu.sync_copy(zbuf, acc.at[pl.ds(s * rpt + j * zrows, zrows), :])
    plsc.subcore_barrier()
    base = s * ept
    def ck(k, carry):
      off = base + k * chunk
      pltpu.sync_copy(src_hbm.at[pl.ds(off, chunk)], sidx)
      pltpu.sync_copy(dst_hbm.at[pl.ds(off, chunk)], didx)
      @pl.when(c == 0)
      def _():
        pltpu.async_copy(ylo_hbm.at[sidx], rows, sem).wait()
      @pl.when(c == 1)
      def _():
        pltpu.async_copy(yhi_hbm.at[sidx], rows, sem).wait()
      pltpu.sync_copy(rows, acc.at[didx], add=True)
      return carry
    lax.fori_loop(0, nch, ck, 0)
    plsc.subcore_barrier()
    @pl.when(c == 0)
    def _():
      pltpu.sync_copy(acc.at[pl.ds(s * rpt, rpt), :],
                      z_hbm.at[0, pl.ds(s * rpt, rpt), :])
    @pl.when(c == 1)
    def _():
      pltpu.sync_copy(acc.at[pl.ds(s * rpt, rpt), :],
                      z_hbm.at[1, pl.ds(s * rpt, rpt), :])

  return pl.kernel(
      body,
      out_type=jax.ShapeDtypeStruct((NSC, n, FW), jnp.float32),
      mesh=_sc_mesh(),
      compiler_params=pltpu.CompilerParams(use_tc_tiling_on_sc=False),
      scratch_types=[
          pltpu.VMEM((chunk,), jnp.int32),
          pltpu.VMEM((chunk,), jnp.int32),
          pltpu.VMEM((chunk, FW), jnp.float32),
          pltpu.VMEM((625, FW), jnp.float32),
          pltpu.VMEM_SHARED((n, FW), jnp.float32),
          pltpu.SemaphoreType.DMA,
      ],
  )


def _make_a1_body(nqo):
  def body(x_ref, dega_ref, degb_ref, w_ref, *outs):
    dinv_ref = outs[-1]
    deg = jnp.sum(dega_ref[0] + degb_ref[0], axis=-1,
                  keepdims=True) * (1.0 / FW) + 1.0
    dinv = lax.rsqrt(deg)
    xw = jnp.dot(x_ref[...], w_ref[...], preferred_element_type=jnp.float32)
    y = xw * dinv
    for q in range(nqo):
      outs[q][...] = y[:, q * FW:(q + 1) * FW]
    dinv_ref[...] = dinv
  return body


def _make_mid_body(nqi, nqo):
  def body(*refs):
    zs = refs[:nqi]
    ys = refs[nqi:2 * nqi]
    dinv_ref, b_ref, w_ref = refs[2 * nqi:2 * nqi + 3]
    outs = refs[2 * nqi + 3:]
    dinv = dinv_ref[...]
    b = b_ref[...]
    w = w_ref[...]
    xw = None
    for q in range(nqi):
      h = dinv * (zs[q][0] + ys[q][...]) + b[:, q * FW:(q + 1) * FW]
      h = jnp.maximum(h, 0.0)
      t = jnp.dot(h, w[q * FW:(q + 1) * FW, :],
                  preferred_element_type=jnp.float32)
      xw = t if xw is None else xw + t
    y = xw * dinv
    for q in range(nqo):
      outs[q][...] = y[:, q * FW:(q + 1) * FW]
  return body


def _make_pool_body(nqi, g):
  def body(*refs):
    zs = refs[:nqi]
    ys = refs[nqi:2 * nqi]
    dinv_ref, b_ref, bt_ref, out_ref, sums, counts = refs[2 * nqi:]
    i = pl.program_id(0)
    @pl.when(i == 0)
    def _():
      sums[...] = jnp.zeros_like(sums)
      counts[...] = jnp.zeros_like(counts)
    dinv = dinv_ref[...]
    b = b_ref[...]
    bt = bt_ref[...]
    gcol = lax.broadcasted_iota(jnp.int32, (bt.shape[0], g), 1)
    m = (bt == gcol).astype(jnp.float32)
    dn = (((0,), (0,)), ((), ()))
    for q in range(nqi):
      h = dinv * (zs[q][0] + ys[q][...]) + b[:, q * FW:(q + 1) * FW]
      sums[:, q * FW:(q + 1) * FW] += lax.dot_general(
          m, h, dn, preferred_element_type=jnp.float32)
    counts[...] += lax.dot_general(
        m, jnp.ones((bt.shape[0], 1), jnp.float32), dn,
        preferred_element_type=jnp.float32)
    @pl.when(i == pl.num_programs(0) - 1)
    def _():
      out_ref[...] = sums[...] / jnp.maximum(counts[...], 1.0)
  return body


def kernel(x, edge_index, batch, W1, b1, W2, b2, W3, b3):
  n, fin = x.shape
  e = edge_index.shape[1]
  h1 = W1.shape[1]
  h2 = W2.shape[1]
  emb = W3.shape[1]
  nq1, nq2, nq3 = h1 // FW, h2 // FW, emb // FW
  g = 64
  blk = 2000
  grid = (n // blk,)

  src, dst = edge_index[0], edge_index[1]
  deg = _make_deg(n, e)(dst)
  prop = _make_prop(n, e)

  zspec = [
      pl.BlockSpec((1, blk, FW), lambda i: (0, i, 0)),
      pl.BlockSpec((1, blk, FW), lambda i: (1, i, 0)),
  ]
  vspec = lambda d: pl.BlockSpec((blk, d), lambda i: (i, 0))
  full = lambda a, bdim: pl.BlockSpec(a, lambda i: (0,) * bdim)
  qshape = [jax.ShapeDtypeStruct((n, FW), jnp.float32)]

  *y1, dinv = pl.pallas_call(
      _make_a1_body(nq1),
      grid=grid,
      in_specs=[vspec(fin)] + zspec + [full((fin, h1), 2)],
      out_specs=[vspec(FW)] * nq1 + [vspec(1)],
      out_shape=qshape * nq1 + [jax.ShapeDtypeStruct((n, 1), jnp.float32)],
  )(x, deg, deg, W1)

  def mid_call(ys, bb, w, nqi, nqo):
    zarrs = []
    for q in range(0, nqi, 2):
      zp = prop(ys[q], ys[q + 1], src, dst)
      zarrs += [zp, zp]
    return pl.pallas_call(
        _make_mid_body(nqi, nqo),
        grid=grid,
        in_specs=(zspec * (nqi // 2) + [vspec(FW)] * nqi
                  + [vspec(1), full((1, nqi * FW), 2),
                     full((nqi * FW, nqo * FW), 2)]),
        out_specs=[vspec(FW)] * nqo,
        out_shape=qshape * nqo,
    )(*zarrs, *ys, dinv, bb.reshape(1, nqi * FW), w)

  y2 = mid_call(list(y1), b1, W2, nq1, nq2)
  y3 = mid_call(list(y2), b2, W3, nq2, nq3)

  z3 = prop(y3[0], y3[1], src, dst)
  out = pl.pallas_call(
      _make_pool_body(nq3, g),
      grid=grid,
      in_specs=(zspec + [vspec(FW)] * nq3
                + [vspec(1), full((1, emb), 2), vspec(1)]),
      out_specs=full((g, emb), 2),
      out_shape=jax.ShapeDtypeStruct((g, emb), jnp.float32),
      scratch_shapes=[
          pltpu.VMEM((g, emb), jnp.float32),
          pltpu.VMEM((g, 1), jnp.float32),
      ],
  )(z3, z3, *y3, dinv, b3.reshape(1, emb),
    batch.reshape(n, 1).astype(jnp.int32))

  return out
